# baseline (speedup 1.0000x reference)
import jax
import jax.numpy as jnp
from jax import lax
from jax.experimental import pallas as pl
from jax.experimental.pallas import tpu as pltpu
from jax.experimental.pallas import tpu_sc as plsc

N = 10000
E = 160000
D = 256
HD = 128
NG = 64
NSUB = 16
RPT = 632
RPT_LAST = N - 15 * RPT

SCH = 125
SROWS = E // SCH
SNCH = (E // NSUB) // SCH
DSTW = 40
DCH = SCH
DROWS = SROWS
DNCH = (E // 32) // DCH

BM = 1000
NB = N // BM

_mesh = plsc.VectorSubcoreMesh(core_axis_name="c", subcore_axis_name="s")


def _via_buf(f_from, f_to, base, sizes, buf, src_is_buf):
    off = 0
    for n in sizes:
        if not src_is_buf:
            pltpu.sync_copy(f_from(base + off, n), buf.at[pl.ds(0, n)])
        pltpu.sync_copy(buf.at[pl.ds(0, n)], f_to(base + off, n))
        off += n


def _staged_rows_copy(s, f_from, f_to, buf, src_is_buf=False):
    @pl.when(s < 15)
    def _():
        _via_buf(f_from, f_to, s * RPT, [80] * 7 + [72], buf, src_is_buf)

    @pl.when(s == 15)
    def _():
        _via_buf(f_from, f_to, 15 * RPT, [80] * 6 + [40], buf, src_is_buf)


def _deg_body(dst_hbm, deg0_hbm, deg1_hbm, dst_v, ones_v, zbuf_v, acc):
    c = lax.axis_index("c")
    s = lax.axis_index("s")
    w = s * 2 + c

    def fill_ones(i, _):
        for k in range(HD // 16):
            ones_v[i, pl.ds(16 * k, 16)] = jnp.ones((16,), jnp.float32)
        return 0

    lax.fori_loop(0, DCH, fill_ones, 0)

    def fill_zero(i, _):
        for k in range(HD // 16):
            zbuf_v[i, pl.ds(16 * k, 16)] = jnp.zeros((16,), jnp.float32)
        return 0

    lax.fori_loop(0, 80, fill_zero, 0)
    _staged_rows_copy(s, lambda o, n: None,
                      lambda o, n: acc.at[pl.ds(o, n)], zbuf_v, src_is_buf=True)
    pltpu.sync_copy(dst_hbm.at[pl.ds(w * DNCH, DNCH)], dst_v)
    plsc.subcore_barrier()

    def step(j, _):
        pltpu.sync_copy(ones_v, acc.at[dst_v.at[j]], add=True)
        return 0

    lax.fori_loop(0, DNCH, step, 0)
    plsc.subcore_barrier()

    @pl.when(c == 0)
    def _():
        _staged_rows_copy(s, lambda o, n: acc.at[pl.ds(o, n)],
                          lambda o, n: deg0_hbm.at[pl.ds(o, n)], zbuf_v)

    @pl.when(c == 1)
    def _():
        _staged_rows_copy(s, lambda o, n: acc.at[pl.ds(o, n)],
                          lambda o, n: deg1_hbm.at[pl.ds(o, n)], zbuf_v)


_deg_kernel = pl.kernel(
    _deg_body,
    out_type=[
        jax.ShapeDtypeStruct((N, HD), jnp.float32),
        jax.ShapeDtypeStruct((N, HD), jnp.float32),
    ],
    mesh=_mesh,
    scratch_types=[
        pltpu.VMEM((DNCH, DCH), jnp.int32),
        pltpu.VMEM((DCH, HD), jnp.float32),
        pltpu.VMEM((80, HD), jnp.float32),
        pltpu.VMEM_SHARED((N, HD), jnp.float32),
    ],
)


def _seg_body(ga_hbm, gb_hbm, src_hbm, dst_hbm, sa_hbm, sb_hbm,
              src_v, dst_w, buf, acc, sem, ssem):
    c = lax.axis_index("c")
    s = lax.axis_index("s")
    pltpu.sync_copy(src_hbm.at[pl.ds(s * SNCH, SNCH)], src_v)

    def run(g_hbm, s_out):
        _staged_rows_copy(s, lambda o, n: g_hbm.at[pl.ds(o, n)],
                          lambda o, n: acc.at[pl.ds(o, n)], buf.at[0])
        plsc.subcore_barrier()

        pltpu.async_copy(g_hbm.at[src_v.at[0]], buf.at[0], sem.at[0])

        def step(j, _):
            p = j % 2

            @pl.when(j >= 1)
            def _():
                pltpu.make_async_copy(
                    buf.at[1 - p], acc.at[dst_w.at[(j - 1) % DSTW]],
                    ssem.at[1 - p]).wait()

            @pl.when(j % DSTW == 0)
            def _():
                jj = pl.multiple_of(j, DSTW)
                pltpu.sync_copy(dst_hbm.at[pl.ds(s * SNCH + jj, DSTW)], dst_w)

            @pl.when(j + 1 < SNCH)
            def _():
                pltpu.async_copy(g_hbm.at[src_v.at[j + 1]], buf.at[1 - p],
                                 sem.at[1 - p])

            pltpu.make_async_copy(g_hbm.at[src_v.at[j]], buf.at[p],
                                  sem.at[p]).wait()
            pltpu.make_async_copy(buf.at[p], acc.at[dst_w.at[j % DSTW]],
                                  ssem.at[p]).start(add=True)
            return 0

        lax.fori_loop(0, SNCH, step, 0)
        pltpu.make_async_copy(buf.at[1], acc.at[dst_w.at[(SNCH - 1) % DSTW]],
                              ssem.at[1]).wait()
        plsc.subcore_barrier()
        _staged_rows_copy(s, lambda o, n: acc.at[pl.ds(o, n)],
                          lambda o, n: s_out.at[pl.ds(o, n)], buf.at[0])

    @pl.when(c == 0)
    def _():
        run(ga_hbm, sa_hbm)

    @pl.when(c == 1)
    def _():
        run(gb_hbm, sb_hbm)


_seg_kernel = pl.kernel(
    _seg_body,
    out_type=[
        jax.ShapeDtypeStruct((N, HD), jnp.float32),
        jax.ShapeDtypeStruct((N, HD), jnp.float32),
    ],
    mesh=_mesh,
    scratch_types=[
        pltpu.VMEM((SNCH, SCH), jnp.int32),
        pltpu.VMEM((DSTW, SCH), jnp.int32),
        pltpu.VMEM((2, SCH, HD), jnp.float32),
        pltpu.VMEM_SHARED((N, HD), jnp.float32),
        pltpu.SemaphoreType.DMA((2,)),
        pltpu.SemaphoreType.DMA((2,)),
    ],
)


def _dinv(d0_ref, d1_ref):
    deg = 1.0 + d0_ref[:, 0:1] + d1_ref[:, 0:1]
    return lax.rsqrt(deg)


def _k1_body(x_ref, we_ref, w0_ref, d0_ref, d1_ref, ga_ref, gb_ref):
    t = jnp.dot(x_ref[...], we_ref[...], preferred_element_type=jnp.float32)
    p = jnp.dot(t, w0_ref[...], preferred_element_type=jnp.float32)
    g = p * _dinv(d0_ref, d1_ref)
    ga_ref[...] = g[:, :HD]
    gb_ref[...] = g[:, HD:]


_wspec = pl.BlockSpec((D, D), lambda i: (0, 0))
_rowspec = pl.BlockSpec((BM, D), lambda i: (i, 0))
_degspec = pl.BlockSpec((BM, HD), lambda i: (i, 0))
_halfspec = pl.BlockSpec((BM, HD), lambda i: (i, 0))
_bspec = pl.BlockSpec((1, D), lambda i: (0, 0))

_k1 = pl.pallas_call(
    _k1_body,
    grid=(NB,),
    in_specs=[_rowspec, _wspec, _wspec, _degspec, _degspec],
    out_specs=[_halfspec, _halfspec],
    out_shape=[jax.ShapeDtypeStruct((N, HD), jnp.float32)] * 2,
)


def _kmid_body(sa_ref, sb_ref, d0_ref, d1_ref, b_ref, w_ref, ga_ref, gb_ref):
    dinv = _dinv(d0_ref, d1_ref)
    sfull = jnp.concatenate([sa_ref[...], sb_ref[...]], axis=1)
    h = jnp.maximum(dinv * sfull + b_ref[...], 0.0)
    p = jnp.dot(h, w_ref[...], preferred_element_type=jnp.float32)
    g = p * dinv
    ga_ref[...] = g[:, :HD]
    gb_ref[...] = g[:, HD:]


_kmid = pl.pallas_call(
    _kmid_body,
    grid=(NB,),
    in_specs=[_halfspec, _halfspec, _degspec, _degspec, _bspec, _wspec],
    out_specs=[_halfspec, _halfspec],
    out_shape=[jax.ShapeDtypeStruct((N, HD), jnp.float32)] * 2,
)


def _kfin_body(sa_ref, sb_ref, d0_ref, d1_ref, b_ref, batch_ref, out_ref, acc):
    i = pl.program_id(0)
    dinv = _dinv(d0_ref, d1_ref)
    sfull = jnp.concatenate([sa_ref[...], sb_ref[...]], axis=1)
    h = dinv * sfull + b_ref[...]
    hh = jnp.concatenate([h, jnp.ones((BM, HD), jnp.float32)], axis=1)
    bb = batch_ref[...].reshape(BM)
    oh = (bb[:, None] == lax.broadcasted_iota(jnp.int32, (BM, NG), 1)
          ).astype(jnp.float32)
    part = lax.dot_general(oh, hh, (((0,), (0,)), ((), ())),
                           preferred_element_type=jnp.float32)

    @pl.when(i == 0)
    def _():
        acc[...] = jnp.zeros((NG, D + HD), jnp.float32)

    acc[...] += part

    @pl.when(i == NB - 1)
    def _():
        out_ref[...] = acc[:, :D] / jnp.maximum(acc[:, D:D + 1], 1.0)


_kfin = pl.pallas_call(
    _kfin_body,
    grid=(NB,),
    in_specs=[_halfspec, _halfspec, _degspec, _degspec, _bspec,
              pl.BlockSpec((1, 1, BM), lambda i: (i, 0, 0))],
    out_specs=pl.BlockSpec((NG, D), lambda i: (0, 0)),
    out_shape=jax.ShapeDtypeStruct((NG, D), jnp.float32),
    scratch_shapes=[pltpu.VMEM((NG, D + HD), jnp.float32)],
)


def kernel(x, edge_index, batch, W_emb, W0, b0, W1, b1, W2, b2, W3, b3):
    src_s = edge_index[0].reshape(SROWS, SCH)
    dst_s = edge_index[1].reshape(SROWS, SCH)
    dst_d = edge_index[1].reshape(DROWS, DCH)
    batch3 = batch.reshape(NB, 1, BM)

    deg0, deg1 = _deg_kernel(dst_d)
    ga, gb = _k1(x, W_emb, W0, deg0, deg1)
    bs = [b0.reshape(1, D), b1.reshape(1, D), b2.reshape(1, D), b3.reshape(1, D)]
    ws = [W1, W2, W3]
    for l in range(3):
        sa, sb = _seg_kernel(ga, gb, src_s, dst_s)
        ga, gb = _kmid(sa, sb, deg0, deg1, bs[l], ws[l])
    sa, sb = _seg_kernel(ga, gb, src_s, dst_s)
    return _kfin(sa, sb, deg0, deg1, bs[3], batch3)


# speedup vs baseline: 1.0567x; 1.0567x over previous
import jax
import jax.numpy as jnp
from jax import lax
from jax.experimental import pallas as pl
from jax.experimental.pallas import tpu as pltpu
from jax.experimental.pallas import tpu_sc as plsc

N = 10000
E = 160000
D = 256
HD = 128
NG = 64
NSUB = 16
RPT = 632
RPT_LAST = N - 15 * RPT

SCH = 125
SROWS = E // SCH
SNCH = (E // NSUB) // SCH
DSTW = 40
DCH = SCH
DROWS = SROWS
DNCH = (E // 32) // DCH
DEGW = 128

BM = 1000
NB = N // BM

_mesh = plsc.VectorSubcoreMesh(core_axis_name="c", subcore_axis_name="s")


def _tile_rows_copy(s, f_from, f_to):
    @pl.when(s < 15)
    def _():
        pltpu.sync_copy(f_from(s * RPT, RPT), f_to(s * RPT, RPT))

    @pl.when(s == 15)
    def _():
        pltpu.sync_copy(f_from(15 * RPT, RPT_LAST), f_to(15 * RPT, RPT_LAST))


def _via_buf(f_from, f_to, base, sizes, buf, src_is_buf):
    off = 0
    for n in sizes:
        if not src_is_buf:
            pltpu.sync_copy(f_from(base + off, n), buf.at[pl.ds(0, n)])
        pltpu.sync_copy(buf.at[pl.ds(0, n)], f_to(base + off, n))
        off += n


def _staged_rows_copy(s, f_from, f_to, buf, src_is_buf=False):
    @pl.when(s < 15)
    def _():
        _via_buf(f_from, f_to, s * RPT, [80] * 7 + [72], buf, src_is_buf)

    @pl.when(s == 15)
    def _():
        _via_buf(f_from, f_to, 15 * RPT, [80] * 6 + [40], buf, src_is_buf)


def _deg_body(dst_hbm, deg0_hbm, deg1_hbm, dst_v, ones_v, zbuf_v, acc):
    c = lax.axis_index("c")
    s = lax.axis_index("s")
    w = s * 2 + c

    def fill_ones(i, _):
        for k in range(DEGW // 16):
            ones_v[i, pl.ds(16 * k, 16)] = jnp.ones((16,), jnp.float32)
        return 0

    lax.fori_loop(0, DCH, fill_ones, 0)

    def fill_zero(i, _):
        for k in range(DEGW // 16):
            zbuf_v[i, pl.ds(16 * k, 16)] = jnp.zeros((16,), jnp.float32)
        return 0

    lax.fori_loop(0, 80, fill_zero, 0)
    _staged_rows_copy(s, lambda o, n: None,
                      lambda o, n: acc.at[pl.ds(o, n)], zbuf_v, src_is_buf=True)
    pltpu.sync_copy(dst_hbm.at[pl.ds(w * DNCH, DNCH)], dst_v)
    plsc.subcore_barrier()

    def step(j, _):
        pltpu.sync_copy(ones_v, acc.at[dst_v.at[j]], add=True)
        return 0

    lax.fori_loop(0, DNCH, step, 0)
    plsc.subcore_barrier()

    @pl.when(c == 0)
    def _():
        _tile_rows_copy(s, lambda o, n: acc.at[pl.ds(o, n)],
                        lambda o, n: deg0_hbm.at[pl.ds(o, n)])

    @pl.when(c == 1)
    def _():
        _tile_rows_copy(s, lambda o, n: acc.at[pl.ds(o, n)],
                        lambda o, n: deg1_hbm.at[pl.ds(o, n)])


_deg_kernel = pl.kernel(
    _deg_body,
    out_type=[
        jax.ShapeDtypeStruct((N, DEGW), jnp.float32),
        jax.ShapeDtypeStruct((N, DEGW), jnp.float32),
    ],
    mesh=_mesh,
    scratch_types=[
        pltpu.VMEM((DNCH, DCH), jnp.int32),
        pltpu.VMEM((DCH, DEGW), jnp.float32),
        pltpu.VMEM((80, DEGW), jnp.float32),
        pltpu.VMEM_SHARED((N, DEGW), jnp.float32),
    ],
)


def _seg_body(ga_hbm, gb_hbm, src_hbm, dst_hbm, sa_hbm, sb_hbm,
              src_v, dst_w, buf, acc, sem, ssem):
    c = lax.axis_index("c")
    s = lax.axis_index("s")
    pltpu.sync_copy(src_hbm.at[pl.ds(s * SNCH, SNCH)], src_v)

    def run(g_hbm, s_out):
        _tile_rows_copy(s, lambda o, n: g_hbm.at[pl.ds(o, n)],
                        lambda o, n: acc.at[pl.ds(o, n)])
        plsc.subcore_barrier()

        pltpu.async_copy(g_hbm.at[src_v.at[0]], buf.at[0], sem.at[0])

        def step(j, _):
            p = j % 2

            @pl.when(j >= 1)
            def _():
                pltpu.make_async_copy(
                    buf.at[1 - p], acc.at[dst_w.at[(j - 1) % DSTW]],
                    ssem.at[1 - p]).wait()

            @pl.when(j % DSTW == 0)
            def _():
                jj = pl.multiple_of(j, DSTW)
                pltpu.sync_copy(dst_hbm.at[pl.ds(s * SNCH + jj, DSTW)], dst_w)

            @pl.when(j + 1 < SNCH)
            def _():
                pltpu.async_copy(g_hbm.at[src_v.at[j + 1]], buf.at[1 - p],
                                 sem.at[1 - p])

            pltpu.make_async_copy(g_hbm.at[src_v.at[j]], buf.at[p],
                                  sem.at[p]).wait()
            pltpu.make_async_copy(buf.at[p], acc.at[dst_w.at[j % DSTW]],
                                  ssem.at[p]).start(add=True)
            return 0

        lax.fori_loop(0, SNCH, step, 0)
        pltpu.make_async_copy(buf.at[1], acc.at[dst_w.at[(SNCH - 1) % DSTW]],
                              ssem.at[1]).wait()
        plsc.subcore_barrier()
        _tile_rows_copy(s, lambda o, n: acc.at[pl.ds(o, n)],
                        lambda o, n: s_out.at[pl.ds(o, n)])

    @pl.when(c == 0)
    def _():
        run(ga_hbm, sa_hbm)

    @pl.when(c == 1)
    def _():
        run(gb_hbm, sb_hbm)


_seg_kernel = pl.kernel(
    _seg_body,
    out_type=[
        jax.ShapeDtypeStruct((N, HD), jnp.float32),
        jax.ShapeDtypeStruct((N, HD), jnp.float32),
    ],
    mesh=_mesh,
    scratch_types=[
        pltpu.VMEM((SNCH, SCH), jnp.int32),
        pltpu.VMEM((DSTW, SCH), jnp.int32),
        pltpu.VMEM((2, SCH, HD), jnp.float32),
        pltpu.VMEM_SHARED((N, HD), jnp.float32),
        pltpu.SemaphoreType.DMA((2,)),
        pltpu.SemaphoreType.DMA((2,)),
    ],
)


def _k1_body(x_ref, we_ref, w0_ref, d0_ref, d1_ref, ga_ref, gb_ref, dv_ref):
    t = jnp.dot(x_ref[...], we_ref[...], preferred_element_type=jnp.float32)
    p = jnp.dot(t, w0_ref[...], preferred_element_type=jnp.float32)
    dinv = lax.rsqrt(1.0 + d0_ref[:, 0:1] + d1_ref[:, 0:1])
    g = p * dinv
    ga_ref[...] = g[:, :HD]
    gb_ref[...] = g[:, HD:]
    dv_ref[...] = jnp.broadcast_to(dinv, (BM, 8))


_wspec = pl.BlockSpec((D, D), lambda i: (0, 0))
_rowspec = pl.BlockSpec((BM, D), lambda i: (i, 0))
_degspec = pl.BlockSpec((BM, DEGW), lambda i: (i, 0))
_halfspec = pl.BlockSpec((BM, HD), lambda i: (i, 0))
_dvspec = pl.BlockSpec((BM, 8), lambda i: (i, 0))
_bspec = pl.BlockSpec((1, D), lambda i: (0, 0))

_k1 = pl.pallas_call(
    _k1_body,
    grid=(NB,),
    in_specs=[_rowspec, _wspec, _wspec, _degspec, _degspec],
    out_specs=[_halfspec, _halfspec, _dvspec],
    out_shape=[jax.ShapeDtypeStruct((N, HD), jnp.float32)] * 2
    + [jax.ShapeDtypeStruct((N, 8), jnp.float32)],
)


def _kmid_body(sa_ref, sb_ref, dv_ref, b_ref, w_ref, ga_ref, gb_ref):
    dinv = dv_ref[:, 0:1]
    sfull = jnp.concatenate([sa_ref[...], sb_ref[...]], axis=1)
    h = jnp.maximum(dinv * sfull + b_ref[...], 0.0)
    p = jnp.dot(h, w_ref[...], preferred_element_type=jnp.float32)
    g = p * dinv
    ga_ref[...] = g[:, :HD]
    gb_ref[...] = g[:, HD:]


_kmid = pl.pallas_call(
    _kmid_body,
    grid=(NB,),
    in_specs=[_halfspec, _halfspec, _dvspec, _bspec, _wspec],
    out_specs=[_halfspec, _halfspec],
    out_shape=[jax.ShapeDtypeStruct((N, HD), jnp.float32)] * 2,
)


def _kfin_body(sa_ref, sb_ref, dv_ref, b_ref, batch_ref, out_ref, acc):
    i = pl.program_id(0)
    dinv = dv_ref[:, 0:1]
    sfull = jnp.concatenate([sa_ref[...], sb_ref[...]], axis=1)
    h = dinv * sfull + b_ref[...]
    hh = jnp.concatenate([h, jnp.ones((BM, HD), jnp.float32)], axis=1)
    bb = batch_ref[...].reshape(BM)
    oh = (bb[:, None] == lax.broadcasted_iota(jnp.int32, (BM, NG), 1)
          ).astype(jnp.float32)
    part = lax.dot_general(oh, hh, (((0,), (0,)), ((), ())),
                           preferred_element_type=jnp.float32)

    @pl.when(i == 0)
    def _():
        acc[...] = jnp.zeros((NG, D + HD), jnp.float32)

    acc[...] += part

    @pl.when(i == NB - 1)
    def _():
        out_ref[...] = acc[:, :D] / jnp.maximum(acc[:, D:D + 1], 1.0)


_kfin = pl.pallas_call(
    _kfin_body,
    grid=(NB,),
    in_specs=[_halfspec, _halfspec, _dvspec, _bspec,
              pl.BlockSpec((1, 1, BM), lambda i: (i, 0, 0))],
    out_specs=pl.BlockSpec((NG, D), lambda i: (0, 0)),
    out_shape=jax.ShapeDtypeStruct((NG, D), jnp.float32),
    scratch_shapes=[pltpu.VMEM((NG, D + HD), jnp.float32)],
)


def kernel(x, edge_index, batch, W_emb, W0, b0, W1, b1, W2, b2, W3, b3):
    src_s = edge_index[0].reshape(SROWS, SCH)
    dst_s = edge_index[1].reshape(SROWS, SCH)
    dst_d = edge_index[1].reshape(DROWS, DCH)
    batch3 = batch.reshape(NB, 1, BM)

    deg0, deg1 = _deg_kernel(dst_d)
    ga, gb, dv = _k1(x, W_emb, W0, deg0, deg1)
    bs = [b0.reshape(1, D), b1.reshape(1, D), b2.reshape(1, D), b3.reshape(1, D)]
    ws = [W1, W2, W3]
    for l in range(3):
        sa, sb = _seg_kernel(ga, gb, src_s, dst_s)
        ga, gb = _kmid(sa, sb, dv, bs[l], ws[l])
    sa, sb = _seg_kernel(ga, gb, src_s, dst_s)
    return _kfin(sa, sb, dv, bs[3], batch3)


# speedup vs baseline: 1.0773x; 1.0195x over previous
import jax
import jax.numpy as jnp
from jax import lax
from jax.experimental import pallas as pl
from jax.experimental.pallas import tpu as pltpu
from jax.experimental.pallas import tpu_sc as plsc

N = 10000
E = 160000
D = 256
HD = 128
NG = 64
NSUB = 16
RPT = 632
RPT_LAST = N - 15 * RPT

SCH = 125
SROWS = E // SCH
SNCH = (E // NSUB) // SCH
DSTW = 40
DCH = SCH
DROWS = SROWS
DNCH = (E // 32) // DCH
DEGW = 128

BM = 2000
NB = N // BM

_mesh = plsc.VectorSubcoreMesh(core_axis_name="c", subcore_axis_name="s")


def _tile_rows_copy(s, f_from, f_to):
    @pl.when(s < 15)
    def _():
        pltpu.sync_copy(f_from(s * RPT, RPT), f_to(s * RPT, RPT))

    @pl.when(s == 15)
    def _():
        pltpu.sync_copy(f_from(15 * RPT, RPT_LAST), f_to(15 * RPT, RPT_LAST))


def _via_buf(f_from, f_to, base, sizes, buf, src_is_buf):
    off = 0
    for n in sizes:
        if not src_is_buf:
            pltpu.sync_copy(f_from(base + off, n), buf.at[pl.ds(0, n)])
        pltpu.sync_copy(buf.at[pl.ds(0, n)], f_to(base + off, n))
        off += n


def _staged_rows_copy(s, f_from, f_to, buf, src_is_buf=False):
    @pl.when(s < 15)
    def _():
        _via_buf(f_from, f_to, s * RPT, [80] * 7 + [72], buf, src_is_buf)

    @pl.when(s == 15)
    def _():
        _via_buf(f_from, f_to, 15 * RPT, [80] * 6 + [40], buf, src_is_buf)


def _deg_body(dst_hbm, deg0_hbm, deg1_hbm, dst_v, ones_v, zbuf_v, acc):
    c = lax.axis_index("c")
    s = lax.axis_index("s")
    w = s * 2 + c

    def fill_ones(i, _):
        for k in range(DEGW // 16):
            ones_v[i, pl.ds(16 * k, 16)] = jnp.ones((16,), jnp.float32)
        return 0

    lax.fori_loop(0, DCH, fill_ones, 0)

    def fill_zero(i, _):
        for k in range(DEGW // 16):
            zbuf_v[i, pl.ds(16 * k, 16)] = jnp.zeros((16,), jnp.float32)
        return 0

    lax.fori_loop(0, 80, fill_zero, 0)
    _staged_rows_copy(s, lambda o, n: None,
                      lambda o, n: acc.at[pl.ds(o, n)], zbuf_v, src_is_buf=True)
    pltpu.sync_copy(dst_hbm.at[pl.ds(w * DNCH, DNCH)], dst_v)
    plsc.subcore_barrier()

    def step(j, _):
        pltpu.sync_copy(ones_v, acc.at[dst_v.at[j]], add=True)
        return 0

    lax.fori_loop(0, DNCH, step, 0)
    plsc.subcore_barrier()

    @pl.when(c == 0)
    def _():
        _tile_rows_copy(s, lambda o, n: acc.at[pl.ds(o, n)],
                        lambda o, n: deg0_hbm.at[pl.ds(o, n)])

    @pl.when(c == 1)
    def _():
        _tile_rows_copy(s, lambda o, n: acc.at[pl.ds(o, n)],
                        lambda o, n: deg1_hbm.at[pl.ds(o, n)])


_deg_kernel = pl.kernel(
    _deg_body,
    out_type=[
        jax.ShapeDtypeStruct((N, DEGW), jnp.float32),
        jax.ShapeDtypeStruct((N, DEGW), jnp.float32),
    ],
    mesh=_mesh,
    scratch_types=[
        pltpu.VMEM((DNCH, DCH), jnp.int32),
        pltpu.VMEM((DCH, DEGW), jnp.float32),
        pltpu.VMEM((80, DEGW), jnp.float32),
        pltpu.VMEM_SHARED((N, DEGW), jnp.float32),
    ],
)


def _seg_body(ga_hbm, gb_hbm, src_hbm, dst_hbm, sa_hbm, sb_hbm,
              src_v, dst_w, buf, acc, sem, ssem):
    c = lax.axis_index("c")
    s = lax.axis_index("s")
    pltpu.sync_copy(src_hbm.at[pl.ds(s * SNCH, SNCH)], src_v)

    def run(g_hbm, s_out):
        pltpu.async_copy(g_hbm.at[src_v.at[0]], buf.at[0], sem.at[0])
        _tile_rows_copy(s, lambda o, n: g_hbm.at[pl.ds(o, n)],
                        lambda o, n: acc.at[pl.ds(o, n)])
        plsc.subcore_barrier()


        def step(j, _):
            p = j % 2

            @pl.when(j >= 1)
            def _():
                pltpu.make_async_copy(
                    buf.at[1 - p], acc.at[dst_w.at[(j - 1) % DSTW]],
                    ssem.at[1 - p]).wait()

            @pl.when(j % DSTW == 0)
            def _():
                jj = pl.multiple_of(j, DSTW)
                pltpu.sync_copy(dst_hbm.at[pl.ds(s * SNCH + jj, DSTW)], dst_w)

            @pl.when(j + 1 < SNCH)
            def _():
                pltpu.async_copy(g_hbm.at[src_v.at[j + 1]], buf.at[1 - p],
                                 sem.at[1 - p])

            pltpu.make_async_copy(g_hbm.at[src_v.at[j]], buf.at[p],
                                  sem.at[p]).wait()
            pltpu.make_async_copy(buf.at[p], acc.at[dst_w.at[j % DSTW]],
                                  ssem.at[p]).start(add=True)
            return 0

        lax.fori_loop(0, SNCH, step, 0)
        pltpu.make_async_copy(buf.at[1], acc.at[dst_w.at[(SNCH - 1) % DSTW]],
                              ssem.at[1]).wait()
        plsc.subcore_barrier()
        _tile_rows_copy(s, lambda o, n: acc.at[pl.ds(o, n)],
                        lambda o, n: s_out.at[pl.ds(o, n)])

    @pl.when(c == 0)
    def _():
        run(ga_hbm, sa_hbm)

    @pl.when(c == 1)
    def _():
        run(gb_hbm, sb_hbm)


_seg_kernel = pl.kernel(
    _seg_body,
    out_type=[
        jax.ShapeDtypeStruct((N, HD), jnp.float32),
        jax.ShapeDtypeStruct((N, HD), jnp.float32),
    ],
    mesh=_mesh,
    scratch_types=[
        pltpu.VMEM((SNCH, SCH), jnp.int32),
        pltpu.VMEM((DSTW, SCH), jnp.int32),
        pltpu.VMEM((2, SCH, HD), jnp.float32),
        pltpu.VMEM_SHARED((N, HD), jnp.float32),
        pltpu.SemaphoreType.DMA((2,)),
        pltpu.SemaphoreType.DMA((2,)),
    ],
)


def _k1_body(x_ref, we_ref, w0_ref, d0_ref, d1_ref, ga_ref, gb_ref, dv_ref):
    t = jnp.dot(x_ref[...], we_ref[...], preferred_element_type=jnp.float32)
    p = jnp.dot(t, w0_ref[...], preferred_element_type=jnp.float32)
    dinv = lax.rsqrt(1.0 + d0_ref[:, 0:1] + d1_ref[:, 0:1])
    g = p * dinv
    ga_ref[...] = g[:, :HD]
    gb_ref[...] = g[:, HD:]
    dv_ref[...] = jnp.broadcast_to(dinv, (BM, 8))


_wspec = pl.BlockSpec((D, D), lambda i: (0, 0))
_rowspec = pl.BlockSpec((BM, D), lambda i: (i, 0))
_degspec = pl.BlockSpec((BM, DEGW), lambda i: (i, 0))
_halfspec = pl.BlockSpec((BM, HD), lambda i: (i, 0))
_dvspec = pl.BlockSpec((BM, 8), lambda i: (i, 0))
_bspec = pl.BlockSpec((1, D), lambda i: (0, 0))

_k1 = pl.pallas_call(
    _k1_body,
    grid=(NB,),
    in_specs=[_rowspec, _wspec, _wspec, _degspec, _degspec],
    out_specs=[_halfspec, _halfspec, _dvspec],
    out_shape=[jax.ShapeDtypeStruct((N, HD), jnp.float32)] * 2
    + [jax.ShapeDtypeStruct((N, 8), jnp.float32)],
)


def _kmid_body(sa_ref, sb_ref, dv_ref, b_ref, w_ref, ga_ref, gb_ref):
    dinv = dv_ref[:, 0:1]
    sfull = jnp.concatenate([sa_ref[...], sb_ref[...]], axis=1)
    h = jnp.maximum(dinv * sfull + b_ref[...], 0.0)
    p = jnp.dot(h, w_ref[...], preferred_element_type=jnp.float32)
    g = p * dinv
    ga_ref[...] = g[:, :HD]
    gb_ref[...] = g[:, HD:]


_kmid = pl.pallas_call(
    _kmid_body,
    grid=(NB,),
    in_specs=[_halfspec, _halfspec, _dvspec, _bspec, _wspec],
    out_specs=[_halfspec, _halfspec],
    out_shape=[jax.ShapeDtypeStruct((N, HD), jnp.float32)] * 2,
)


def _kfin_body(sa_ref, sb_ref, dv_ref, b_ref, batch_ref, out_ref, acc):
    i = pl.program_id(0)
    dinv = dv_ref[:, 0:1]
    sfull = jnp.concatenate([sa_ref[...], sb_ref[...]], axis=1)
    h = dinv * sfull + b_ref[...]
    hh = jnp.concatenate([h, jnp.ones((BM, HD), jnp.float32)], axis=1)
    bb = batch_ref[...].reshape(BM)
    oh = (bb[:, None] == lax.broadcasted_iota(jnp.int32, (BM, NG), 1)
          ).astype(jnp.float32)
    part = lax.dot_general(oh, hh, (((0,), (0,)), ((), ())),
                           preferred_element_type=jnp.float32)

    @pl.when(i == 0)
    def _():
        acc[...] = jnp.zeros((NG, D + HD), jnp.float32)

    acc[...] += part

    @pl.when(i == NB - 1)
    def _():
        out_ref[...] = acc[:, :D] / jnp.maximum(acc[:, D:D + 1], 1.0)


_kfin = pl.pallas_call(
    _kfin_body,
    grid=(NB,),
    in_specs=[_halfspec, _halfspec, _dvspec, _bspec,
              pl.BlockSpec((1, 1, BM), lambda i: (i, 0, 0))],
    out_specs=pl.BlockSpec((NG, D), lambda i: (0, 0)),
    out_shape=jax.ShapeDtypeStruct((NG, D), jnp.float32),
    scratch_shapes=[pltpu.VMEM((NG, D + HD), jnp.float32)],
)


def kernel(x, edge_index, batch, W_emb, W0, b0, W1, b1, W2, b2, W3, b3):
    src_s = edge_index[0].reshape(SROWS, SCH)
    dst_s = edge_index[1].reshape(SROWS, SCH)
    dst_d = edge_index[1].reshape(DROWS, DCH)
    batch3 = batch.reshape(NB, 1, BM)

    deg0, deg1 = _deg_kernel(dst_d)
    ga, gb, dv = _k1(x, W_emb, W0, deg0, deg1)
    bs = [b0.reshape(1, D), b1.reshape(1, D), b2.reshape(1, D), b3.reshape(1, D)]
    ws = [W1, W2, W3]
    for l in range(3):
        sa, sb = _seg_kernel(ga, gb, src_s, dst_s)
        ga, gb = _kmid(sa, sb, dv, bs[l], ws[l])
    sa, sb = _seg_kernel(ga, gb, src_s, dst_s)
    return _kfin(sa, sb, dv, bs[3], batch3)


# speedup vs baseline: 1.0798x; 1.0023x over previous
import jax
import jax.numpy as jnp
from jax import lax
from jax.experimental import pallas as pl
from jax.experimental.pallas import tpu as pltpu
from jax.experimental.pallas import tpu_sc as plsc

N = 10000
E = 160000
D = 256
HD = 128
NG = 64
NSUB = 16
RPT = 632
RPT_LAST = N - 15 * RPT

SCH = 125
SROWS = E // SCH
SNCH = (E // NSUB) // SCH
DSTW = 40
DCH = SCH
DROWS = SROWS
DNCH = (E // 32) // DCH
DEGW = 128

BM = 2000
NB = N // BM

_mesh = plsc.VectorSubcoreMesh(core_axis_name="c", subcore_axis_name="s")


def _tile_rows_copy(s, f_from, f_to):
    @pl.when(s < 15)
    def _():
        pltpu.sync_copy(f_from(s * RPT, RPT), f_to(s * RPT, RPT))

    @pl.when(s == 15)
    def _():
        pltpu.sync_copy(f_from(15 * RPT, RPT_LAST), f_to(15 * RPT, RPT_LAST))


def _via_buf(f_from, f_to, base, sizes, buf, src_is_buf):
    off = 0
    for n in sizes:
        if not src_is_buf:
            pltpu.sync_copy(f_from(base + off, n), buf.at[pl.ds(0, n)])
        pltpu.sync_copy(buf.at[pl.ds(0, n)], f_to(base + off, n))
        off += n


def _staged_rows_copy(s, f_from, f_to, buf, src_is_buf=False):
    @pl.when(s < 15)
    def _():
        _via_buf(f_from, f_to, s * RPT, [80] * 7 + [72], buf, src_is_buf)

    @pl.when(s == 15)
    def _():
        _via_buf(f_from, f_to, 15 * RPT, [80] * 6 + [40], buf, src_is_buf)


def _deg_body(dst_hbm, deg0_hbm, deg1_hbm, dst_v, ones_v, zbuf_v, acc):
    c = lax.axis_index("c")
    s = lax.axis_index("s")
    w = s * 2 + c

    def fill_ones(i, _):
        for k in range(DEGW // 16):
            ones_v[i, pl.ds(16 * k, 16)] = jnp.ones((16,), jnp.float32)
        return 0

    lax.fori_loop(0, DCH, fill_ones, 0)

    def fill_zero(i, _):
        for k in range(DEGW // 16):
            zbuf_v[i, pl.ds(16 * k, 16)] = jnp.zeros((16,), jnp.float32)
        return 0

    lax.fori_loop(0, 80, fill_zero, 0)
    _staged_rows_copy(s, lambda o, n: None,
                      lambda o, n: acc.at[pl.ds(o, n)], zbuf_v, src_is_buf=True)
    pltpu.sync_copy(dst_hbm.at[pl.ds(w * DNCH, DNCH)], dst_v)
    plsc.subcore_barrier()

    def step(j, _):
        pltpu.sync_copy(ones_v, acc.at[dst_v.at[j]], add=True)
        return 0

    lax.fori_loop(0, DNCH, step, 0)
    plsc.subcore_barrier()

    @pl.when(c == 0)
    def _():
        _tile_rows_copy(s, lambda o, n: acc.at[pl.ds(o, n)],
                        lambda o, n: deg0_hbm.at[pl.ds(o, n)])

    @pl.when(c == 1)
    def _():
        _tile_rows_copy(s, lambda o, n: acc.at[pl.ds(o, n)],
                        lambda o, n: deg1_hbm.at[pl.ds(o, n)])


_deg_kernel = pl.kernel(
    _deg_body,
    out_type=[
        jax.ShapeDtypeStruct((N, DEGW), jnp.float32),
        jax.ShapeDtypeStruct((N, DEGW), jnp.float32),
    ],
    mesh=_mesh,
    scratch_types=[
        pltpu.VMEM((DNCH, DCH), jnp.int32),
        pltpu.VMEM((DCH, DEGW), jnp.float32),
        pltpu.VMEM((80, DEGW), jnp.float32),
        pltpu.VMEM_SHARED((N, DEGW), jnp.float32),
    ],
)


def _seg_body(ga_hbm, gb_hbm, src_hbm, dst_hbm, sa_hbm, sb_hbm,
              src_v, dst_w, buf, acc, sem, ssem):
    c = lax.axis_index("c")
    s = lax.axis_index("s")
    pltpu.sync_copy(src_hbm.at[pl.ds(s * SNCH, SNCH)], src_v)

    def run(g_hbm, s_out):
        pltpu.async_copy(g_hbm.at[src_v.at[0]], buf.at[0], sem.at[0])
        _tile_rows_copy(s, lambda o, n: g_hbm.at[pl.ds(o, n)],
                        lambda o, n: acc.at[pl.ds(o, n)])
        plsc.subcore_barrier()


        def step(j, _):
            p = j % 2

            @pl.when(j >= 1)
            def _():
                pltpu.make_async_copy(
                    buf.at[1 - p], acc.at[dst_w.at[(j - 1) % DSTW]],
                    ssem.at[1 - p]).wait()

            @pl.when(j % DSTW == 0)
            def _():
                jj = pl.multiple_of(j, DSTW)
                pltpu.sync_copy(dst_hbm.at[pl.ds(s * SNCH + jj, DSTW)], dst_w)

            @pl.when(j + 1 < SNCH)
            def _():
                pltpu.async_copy(g_hbm.at[src_v.at[j + 1]], buf.at[1 - p],
                                 sem.at[1 - p])

            pltpu.make_async_copy(g_hbm.at[src_v.at[j]], buf.at[p],
                                  sem.at[p]).wait()
            pltpu.make_async_copy(buf.at[p], acc.at[dst_w.at[j % DSTW]],
                                  ssem.at[p]).start(add=True)
            return 0

        lax.fori_loop(0, SNCH, step, 0)
        pltpu.make_async_copy(buf.at[1], acc.at[dst_w.at[(SNCH - 1) % DSTW]],
                              ssem.at[1]).wait()
        plsc.subcore_barrier()
        _tile_rows_copy(s, lambda o, n: acc.at[pl.ds(o, n)],
                        lambda o, n: s_out.at[pl.ds(o, n)])

    @pl.when(c == 0)
    def _():
        run(ga_hbm, sa_hbm)

    @pl.when(c == 1)
    def _():
        run(gb_hbm, sb_hbm)


_seg_kernel = pl.kernel(
    _seg_body,
    out_type=[
        jax.ShapeDtypeStruct((N, HD), jnp.float32),
        jax.ShapeDtypeStruct((N, HD), jnp.float32),
    ],
    mesh=_mesh,
    scratch_types=[
        pltpu.VMEM((SNCH, SCH), jnp.int32),
        pltpu.VMEM((DSTW, SCH), jnp.int32),
        pltpu.VMEM((2, SCH, HD), jnp.float32),
        pltpu.VMEM_SHARED((N, HD), jnp.float32),
        pltpu.SemaphoreType.DMA((2,)),
        pltpu.SemaphoreType.DMA((2,)),
    ],
)


def _k1a_body(x_ref, we_ref, w0_ref, pa_ref, pb_ref):
    t = jnp.dot(x_ref[...], we_ref[...], preferred_element_type=jnp.float32)
    p = jnp.dot(t, w0_ref[...], preferred_element_type=jnp.float32)
    pa_ref[...] = p[:, :HD]
    pb_ref[...] = p[:, HD:]


def _k1b_body(pa_ref, pb_ref, d0_ref, d1_ref, ga_ref, gb_ref, dv_ref):
    dinv = lax.rsqrt(1.0 + d0_ref[:, 0:1] + d1_ref[:, 0:1])
    ga_ref[...] = pa_ref[...] * dinv
    gb_ref[...] = pb_ref[...] * dinv
    dv_ref[...] = jnp.broadcast_to(dinv, (BM, 8))


_wspec = pl.BlockSpec((D, D), lambda i: (0, 0))
_rowspec = pl.BlockSpec((BM, D), lambda i: (i, 0))
_degspec = pl.BlockSpec((BM, DEGW), lambda i: (i, 0))
_halfspec = pl.BlockSpec((BM, HD), lambda i: (i, 0))
_dvspec = pl.BlockSpec((BM, 8), lambda i: (i, 0))
_bspec = pl.BlockSpec((1, D), lambda i: (0, 0))

_k1a = pl.pallas_call(
    _k1a_body,
    grid=(NB,),
    in_specs=[_rowspec, _wspec, _wspec],
    out_specs=[_halfspec, _halfspec],
    out_shape=[jax.ShapeDtypeStruct((N, HD), jnp.float32)] * 2,
)

_k1b = pl.pallas_call(
    _k1b_body,
    grid=(NB,),
    in_specs=[_halfspec, _halfspec, _degspec, _degspec],
    out_specs=[_halfspec, _halfspec, _dvspec],
    out_shape=[jax.ShapeDtypeStruct((N, HD), jnp.float32)] * 2
    + [jax.ShapeDtypeStruct((N, 8), jnp.float32)],
)


def _kmid_body(sa_ref, sb_ref, dv_ref, b_ref, w_ref, ga_ref, gb_ref):
    dinv = dv_ref[:, 0:1]
    sfull = jnp.concatenate([sa_ref[...], sb_ref[...]], axis=1)
    h = jnp.maximum(dinv * sfull + b_ref[...], 0.0)
    p = jnp.dot(h, w_ref[...], preferred_element_type=jnp.float32)
    g = p * dinv
    ga_ref[...] = g[:, :HD]
    gb_ref[...] = g[:, HD:]


_kmid = pl.pallas_call(
    _kmid_body,
    grid=(NB,),
    in_specs=[_halfspec, _halfspec, _dvspec, _bspec, _wspec],
    out_specs=[_halfspec, _halfspec],
    out_shape=[jax.ShapeDtypeStruct((N, HD), jnp.float32)] * 2,
)


def _kfin_body(sa_ref, sb_ref, dv_ref, b_ref, batch_ref, out_ref, acc):
    i = pl.program_id(0)
    dinv = dv_ref[:, 0:1]
    sfull = jnp.concatenate([sa_ref[...], sb_ref[...]], axis=1)
    h = dinv * sfull + b_ref[...]
    hh = jnp.concatenate([h, jnp.ones((BM, HD), jnp.float32)], axis=1)
    bb = batch_ref[...].reshape(BM)
    oh = (bb[:, None] == lax.broadcasted_iota(jnp.int32, (BM, NG), 1)
          ).astype(jnp.float32)
    part = lax.dot_general(oh, hh, (((0,), (0,)), ((), ())),
                           preferred_element_type=jnp.float32)

    @pl.when(i == 0)
    def _():
        acc[...] = jnp.zeros((NG, D + HD), jnp.float32)

    acc[...] += part

    @pl.when(i == NB - 1)
    def _():
        out_ref[...] = acc[:, :D] / jnp.maximum(acc[:, D:D + 1], 1.0)


_kfin = pl.pallas_call(
    _kfin_body,
    grid=(NB,),
    in_specs=[_halfspec, _halfspec, _dvspec, _bspec,
              pl.BlockSpec((1, 1, BM), lambda i: (i, 0, 0))],
    out_specs=pl.BlockSpec((NG, D), lambda i: (0, 0)),
    out_shape=jax.ShapeDtypeStruct((NG, D), jnp.float32),
    scratch_shapes=[pltpu.VMEM((NG, D + HD), jnp.float32)],
)


def kernel(x, edge_index, batch, W_emb, W0, b0, W1, b1, W2, b2, W3, b3):
    src_s = edge_index[0].reshape(SROWS, SCH)
    dst_s = edge_index[1].reshape(SROWS, SCH)
    dst_d = edge_index[1].reshape(DROWS, DCH)
    batch3 = batch.reshape(NB, 1, BM)

    deg0, deg1 = _deg_kernel(dst_d)
    pa, pb = _k1a(x, W_emb, W0)
    ga, gb, dv = _k1b(pa, pb, deg0, deg1)
    bs = [b0.reshape(1, D), b1.reshape(1, D), b2.reshape(1, D), b3.reshape(1, D)]
    ws = [W1, W2, W3]
    for l in range(3):
        sa, sb = _seg_kernel(ga, gb, src_s, dst_s)
        ga, gb = _kmid(sa, sb, dv, bs[l], ws[l])
    sa, sb = _seg_kernel(ga, gb, src_s, dst_s)
    return _kfin(sa, sb, dv, bs[3], batch3)
